# baseline (speedup 1.0000x reference)
import functools
import jax
import jax.numpy as jnp
from jax import lax
from jax.experimental import pallas as pl
from jax.experimental.pallas import tpu as pltpu
from jax.experimental.pallas import tpu_sc as plsc

H = 128
NE = 10000
NEP = 10240
NR = 460
NRP = 512
E = 320000
NC = 2
NS = 16
NW = NC * NS
K = 40
NB = 8
EPW = 10240
EP = EPW * NW
CH = EPW // K
SLOPE = (1.0 / 8.0 + 1.0 / 3.0) / 2.0

_mesh = plsc.VectorSubcoreMesh(core_axis_name="c", subcore_axis_name="s",
                               num_cores=NC, num_subcores=NS)


@functools.partial(
    pl.kernel,
    out_type=[jax.ShapeDtypeStruct((NC, NEP, H), jnp.float32),
              jax.ShapeDtypeStruct((NC, NRP, H), jnp.float32)],
    mesh=_mesh,
    scratch_types=[
        pltpu.VMEM((2, 3, K), jnp.int32),
        pltpu.VMEM((K, H), jnp.float32),
        pltpu.VMEM_SHARED((NEP, H), jnp.float32),
        pltpu.VMEM_SHARED((NRP, H), jnp.float32),
        pltpu.SemaphoreType.DMA,
        pltpu.SemaphoreType.DMA,
    ],
)
def _counts(edges3_hbm, zeros_hbm, ones_hbm, out_deg, out_rc,
            idx_v, ones_v, acc_deg, acc_rc, sem0, sem1):
    cid = lax.axis_index("c")
    sid = lax.axis_index("s")
    wid = sid * NC + cid
    rpt = NEP // NS
    rpr = NRP // NS
    pltpu.sync_copy(zeros_hbm.at[pl.ds(sid * rpt, rpt)],
                    acc_deg.at[pl.ds(sid * rpt, rpt)])
    pltpu.sync_copy(zeros_hbm.at[pl.ds(sid * rpr, rpr)],
                    acc_rc.at[pl.ds(sid * rpr, rpr)])
    pltpu.sync_copy(ones_hbm, ones_v)
    plsc.subcore_barrier()
    cbase = wid * CH
    sems = (sem0, sem1)

    def start(c, b):
        pltpu.async_copy(edges3_hbm.at[cbase + c], idx_v.at[b], sems[b])

    def finish(b):
        pltpu.make_async_copy(edges3_hbm.at[cbase], idx_v.at[b], sems[b]).wait()
        pltpu.sync_copy(ones_v, acc_deg.at[idx_v.at[b, 1]], add=True)
        pltpu.sync_copy(ones_v, acc_rc.at[idx_v.at[b, 2]], add=True)

    start(0, 0)

    @pl.loop(0, CH, step=2)
    def _(c):
        start(c + 1, 1)
        finish(0)

        @pl.when(c + 2 < CH)
        def _():
            start(c + 2, 0)

        finish(1)

    plsc.subcore_barrier()
    pltpu.sync_copy(acc_deg.at[pl.ds(sid * rpt, rpt)],
                    out_deg.at[cid, pl.ds(sid * rpt, rpt)])
    pltpu.sync_copy(acc_rc.at[pl.ds(sid * rpr, rpr)],
                    out_rc.at[cid, pl.ds(sid * rpr, rpr)])


@functools.partial(
    pl.kernel,
    out_type=[jax.ShapeDtypeStruct((NC, NEP, H), jnp.float32),
              jax.ShapeDtypeStruct((NC, NRP, H), jnp.float32)],
    mesh=_mesh,
    scratch_types=[
        pltpu.VMEM((NB, 3, K), jnp.int32),
        pltpu.VMEM((NB, K, H), jnp.float32),
        pltpu.VMEM_SHARED((NEP, H), jnp.float32),
        pltpu.VMEM_SHARED((NRP, H), jnp.float32),
    ] + [pltpu.SemaphoreType.DMA] * NB,
)
def _edge_pass(tab_hbm, edges3_hbm, zeros_hbm, out_dst, out_rel,
               idx_v, rows_v, acc_dst, acc_rel, *sems):
    cid = lax.axis_index("c")
    sid = lax.axis_index("s")
    wid = sid * NC + cid
    rpt = NEP // NS
    rpr = NRP // NS
    pltpu.sync_copy(zeros_hbm.at[pl.ds(sid * rpt, rpt)],
                    acc_dst.at[pl.ds(sid * rpt, rpt)])
    pltpu.sync_copy(zeros_hbm.at[pl.ds(sid * rpr, rpr)],
                    acc_rel.at[pl.ds(sid * rpr, rpr)])
    plsc.subcore_barrier()
    cbase = wid * CH

    def start(c, b):
        pltpu.sync_copy(edges3_hbm.at[cbase + c], idx_v.at[b])
        pltpu.async_copy(tab_hbm.at[idx_v.at[b, 0]], rows_v.at[b], sems[b])

    def finish(b):
        pltpu.make_async_copy(tab_hbm.at[idx_v.at[b, 0]], rows_v.at[b],
                              sems[b]).wait()
        pltpu.sync_copy(rows_v.at[b], acc_dst.at[idx_v.at[b, 1]], add=True)
        pltpu.sync_copy(rows_v.at[b], acc_rel.at[idx_v.at[b, 2]], add=True)

    for j in range(NB - 1):
        start(j, j)

    @pl.loop(0, CH, step=NB)
    def _(c):
        for b in range(NB):
            nc = c + b + NB - 1
            nb = (b + NB - 1) % NB

            @pl.when(nc < CH)
            def _():
                start(nc, nb)

            finish(b)

    plsc.subcore_barrier()
    pltpu.sync_copy(acc_dst.at[pl.ds(sid * rpt, rpt)],
                    out_dst.at[cid, pl.ds(sid * rpt, rpt)])
    pltpu.sync_copy(acc_rel.at[pl.ds(sid * rpr, rpr)],
                    out_rel.at[cid, pl.ds(sid * rpr, rpr)])


@functools.partial(
    pl.kernel,
    out_type=[jax.ShapeDtypeStruct((NC, NEP, H), jnp.float32)],
    mesh=_mesh,
    scratch_types=[
        pltpu.VMEM((NB, 3, K), jnp.int32),
        pltpu.VMEM((NB, K, H), jnp.float32),
        pltpu.VMEM_SHARED((NEP, H), jnp.float32),
    ] + [pltpu.SemaphoreType.DMA] * NB,
)
def _rel_pass(r_hbm, edges3_hbm, zeros_hbm, out_dst,
              idx_v, rows_v, acc_dst, *sems):
    cid = lax.axis_index("c")
    sid = lax.axis_index("s")
    wid = sid * NC + cid
    rpt = NEP // NS
    pltpu.sync_copy(zeros_hbm.at[pl.ds(sid * rpt, rpt)],
                    acc_dst.at[pl.ds(sid * rpt, rpt)])
    plsc.subcore_barrier()
    cbase = wid * CH

    def start(c, b):
        pltpu.sync_copy(edges3_hbm.at[cbase + c], idx_v.at[b])
        pltpu.async_copy(r_hbm.at[idx_v.at[b, 2]], rows_v.at[b], sems[b])

    def finish(b):
        pltpu.make_async_copy(r_hbm.at[idx_v.at[b, 2]], rows_v.at[b],
                              sems[b]).wait()
        pltpu.sync_copy(rows_v.at[b], acc_dst.at[idx_v.at[b, 1]], add=True)

    for j in range(NB - 1):
        start(j, j)

    @pl.loop(0, CH, step=NB)
    def _(c):
        for b in range(NB):
            nc = c + b + NB - 1
            nb = (b + NB - 1) % NB

            @pl.when(nc < CH)
            def _():
                start(nc, nb)

            finish(b)

    plsc.subcore_barrier()
    pltpu.sync_copy(acc_dst.at[pl.ds(sid * rpt, rpt)],
                    out_dst.at[cid, pl.ds(sid * rpt, rpt)])


def _norm_body(x_ref, o_ref):
    x = x_ref[...]
    n = jnp.sqrt(jnp.sum(x * x, axis=1, keepdims=True)) + 1e-12
    o_ref[...] = x / n


def _l2norm_rows(x):
    rows = x.shape[0]
    blk = 1024 if rows % 1024 == 0 else rows
    spec = pl.BlockSpec((blk, H), lambda i: (i, 0))
    return pl.pallas_call(
        _norm_body,
        grid=(rows // blk,),
        in_specs=[spec],
        out_specs=spec,
        out_shape=jax.ShapeDtypeStruct((rows, H), jnp.float32),
    )(x)


def _gru(r_agg, r, er, wih, whh, bih, bhh):
    x = jnp.concatenate([r_agg, er], axis=1)
    gi = jnp.dot(x, wih, preferred_element_type=jnp.float32) + bih
    gh = jnp.dot(r, whh, preferred_element_type=jnp.float32) + bhh
    rg = jax.nn.sigmoid(gi[:, 0:H] + gh[:, 0:H])
    zg = jax.nn.sigmoid(gi[:, H:2 * H] + gh[:, H:2 * H])
    ng = jnp.tanh(gi[:, 2 * H:3 * H] + rg * gh[:, 2 * H:3 * H])
    r_new = (1.0 - zg) * ng + zg * r
    n = jnp.sqrt(jnp.sum(r_new * r_new, axis=1, keepdims=True)) + 1e-12
    return r_new / n


def _rel_first_body(sr0_ref, sr1_ref, rc0_ref, rc1_ref, r_ref, er_ref,
                    wih_ref, whh_ref, bih_ref, bhh_ref, out_r, out_invrc):
    invrc = 1.0 / jnp.maximum(rc0_ref[...][:, 0:1] + rc1_ref[...][:, 0:1], 1.0)
    r_agg = (sr0_ref[...] + sr1_ref[...]) * invrc
    rn = _gru(r_agg, r_ref[...], er_ref[...], wih_ref[...],
              whh_ref[...], bih_ref[...], bhh_ref[...])
    out_r[...] = rn
    out_invrc[...] = jnp.broadcast_to(invrc, (NRP, H))


def _rel_rest_body(sr0_ref, sr1_ref, invrc_ref, r_ref, er_ref, wih_ref,
                   whh_ref, bih_ref, bhh_ref, out_r):
    r_agg = (sr0_ref[...] + sr1_ref[...]) * invrc_ref[...]
    out_r[...] = _gru(r_agg, r_ref[...], er_ref[...], wih_ref[...],
                      whh_ref[...], bih_ref[...], bhh_ref[...])


def _rel_first(sr0, sr1, rc0, rc1, r, er, wih_t, whh_t, bih, bhh):
    return pl.pallas_call(
        _rel_first_body,
        out_shape=[jax.ShapeDtypeStruct((NRP, H), jnp.float32),
                   jax.ShapeDtypeStruct((NRP, H), jnp.float32)],
    )(sr0, sr1, rc0, rc1, r, er, wih_t, whh_t, bih, bhh)


def _rel_rest(sr0, sr1, invrc, r, er, wih_t, whh_t, bih, bhh):
    return pl.pallas_call(
        _rel_rest_body,
        out_shape=jax.ShapeDtypeStruct((NRP, H), jnp.float32),
    )(sr0, sr1, invrc, r, er, wih_t, whh_t, bih, bhh)


def _entity(x, h, wn, wl, tw, tb):
    agg = jnp.dot(x, wn, preferred_element_type=jnp.float32)
    loop = jnp.dot(h, wl, preferred_element_type=jnp.float32)
    t = agg + loop
    cur = jnp.where(t >= 0, t, t * SLOPE)
    n = jnp.sqrt(jnp.sum(cur * cur, axis=1, keepdims=True)) + 1e-12
    cur = cur / n
    tg = jax.nn.sigmoid(jnp.dot(cur, tw, preferred_element_type=jnp.float32) + tb)
    return tg * cur + (1.0 - tg) * h


def _h_first_body(sh0_ref, sh1_ref, sr0_ref, sr1_ref, d0_ref, d1_ref, h_ref,
                  wn_ref, wl_ref, tw_ref, tb_ref, out_h, out_invdeg):
    invdeg = 1.0 / jnp.maximum(d0_ref[...][:, 0:1] + d1_ref[...][:, 0:1], 1.0)
    s = sh0_ref[...] + sh1_ref[...] + sr0_ref[...] + sr1_ref[...]
    hn = _entity(s * invdeg, h_ref[...], wn_ref[...], wl_ref[...],
                 tw_ref[...], tb_ref[...])
    out_h[...] = hn
    out_invdeg[...] = jnp.broadcast_to(invdeg, (out_invdeg.shape[0], H))


def _h_rest_body(sh0_ref, sh1_ref, sr0_ref, sr1_ref, invdeg_ref, h_ref,
                 wn_ref, wl_ref, tw_ref, tb_ref, out_h):
    s = sh0_ref[...] + sh1_ref[...] + sr0_ref[...] + sr1_ref[...]
    out_h[...] = _entity(s * invdeg_ref[...], h_ref[...], wn_ref[...],
                         wl_ref[...], tw_ref[...], tb_ref[...])


def _h_first(sh0, sh1, sr0, sr1, d0, d1, h, wn, wl, tw, tb):
    blk = 1024
    row_spec = pl.BlockSpec((blk, H), lambda i: (i, 0))
    w_spec = pl.BlockSpec((H, H), lambda i: (0, 0))
    b_spec = pl.BlockSpec((1, H), lambda i: (0, 0))
    return pl.pallas_call(
        _h_first_body,
        grid=(NEP // blk,),
        in_specs=[row_spec, row_spec, row_spec, row_spec, row_spec, row_spec,
                  row_spec, w_spec, w_spec, w_spec, b_spec],
        out_specs=[row_spec, row_spec],
        out_shape=[jax.ShapeDtypeStruct((NEP, H), jnp.float32),
                   jax.ShapeDtypeStruct((NEP, H), jnp.float32)],
    )(sh0, sh1, sr0, sr1, d0, d1, h, wn, wl, tw, tb)


def _h_rest(sh0, sh1, sr0, sr1, invdeg, h, wn, wl, tw, tb):
    blk = 1024
    row_spec = pl.BlockSpec((blk, H), lambda i: (i, 0))
    w_spec = pl.BlockSpec((H, H), lambda i: (0, 0))
    b_spec = pl.BlockSpec((1, H), lambda i: (0, 0))
    return pl.pallas_call(
        _h_rest_body,
        grid=(NEP // blk,),
        in_specs=[row_spec, row_spec, row_spec, row_spec, row_spec, row_spec,
                  w_spec, w_spec, w_spec, b_spec],
        out_specs=row_spec,
        out_shape=jax.ShapeDtypeStruct((NEP, H), jnp.float32),
    )(sh0, sh1, sr0, sr1, invdeg, h, wn, wl, tw, tb)


def kernel(edge_index, etype, dynamic_emb, emb_rel, w_neigh, w_loop,
           time_gate_weight, time_gate_bias, gru_w_ih, gru_w_hh, gru_b_ih, gru_b_hh):
    pad = EP - E
    src_p = jnp.concatenate([jnp.asarray(edge_index[0], jnp.int32),
                             jnp.zeros((pad,), jnp.int32)])
    dst_p = jnp.concatenate([jnp.asarray(edge_index[1], jnp.int32),
                             jnp.full((pad,), NE, jnp.int32)])
    et_p = jnp.concatenate([jnp.asarray(etype, jnp.int32),
                            jnp.full((pad,), NR, jnp.int32)])
    edges3 = jnp.stack([src_p.reshape(NW, CH, K), dst_p.reshape(NW, CH, K),
                        et_p.reshape(NW, CH, K)], axis=2).reshape(NW * CH, 3, K)

    emb_pad = jnp.concatenate(
        [dynamic_emb, jnp.zeros((NEP - NE, H), jnp.float32)], axis=0)
    er_p = jnp.concatenate(
        [emb_rel, jnp.zeros((NRP - NR, H), jnp.float32)], axis=0)

    zeros_h = jnp.zeros((NEP, H), jnp.float32)
    ones_h = jnp.ones((K, H), jnp.float32)

    wih_t = jnp.transpose(gru_w_ih)
    whh_t = jnp.transpose(gru_w_hh)
    bih = jnp.reshape(gru_b_ih, (1, 3 * H))
    bhh = jnp.reshape(gru_b_hh, (1, 3 * H))
    tb = jnp.reshape(time_gate_bias, (1, H))

    deg_p, rc_p = _counts(edges3, zeros_h, ones_h)
    h = _l2norm_rows(emb_pad)

    sdh, srel = _edge_pass(h, edges3, zeros_h)
    r, invrc = _rel_first(srel[0], srel[1], rc_p[0], rc_p[1], er_p, er_p,
                          wih_t, whh_t, bih, bhh)
    (sdr,) = _rel_pass(r, edges3, zeros_h)
    h, invdeg = _h_first(sdh[0], sdh[1], sdr[0], sdr[1],
                         deg_p[0], deg_p[1], h,
                         w_neigh, w_loop, time_gate_weight, tb)

    for _ in range(2):
        sdh, srel = _edge_pass(h, edges3, zeros_h)
        r = _rel_rest(srel[0], srel[1], invrc, r, er_p,
                      wih_t, whh_t, bih, bhh)
        (sdr,) = _rel_pass(r, edges3, zeros_h)
        h = _h_rest(sdh[0], sdh[1], sdr[0], sdr[1], invdeg, h,
                    w_neigh, w_loop, time_gate_weight, tb)
    return h[:NE]


# speedup vs baseline: 1.0560x; 1.0560x over previous
import functools
import jax
import jax.numpy as jnp
from jax import lax
from jax.experimental import pallas as pl
from jax.experimental.pallas import tpu as pltpu
from jax.experimental.pallas import tpu_sc as plsc

H = 128
NE = 10000
NEP = 10240
NR = 460
NRP = 512
E = 320000
NC = 2
NS = 16
NW = NC * NS
K = 80
NB = 4
EPW = 10240
EP = EPW * NW
CH = EPW // K
SLOPE = (1.0 / 8.0 + 1.0 / 3.0) / 2.0

_mesh = plsc.VectorSubcoreMesh(core_axis_name="c", subcore_axis_name="s",
                               num_cores=NC, num_subcores=NS)


def _edge_body(tab_hbm, edges3_hbm, zeros_hbm, out_dst, out_rel,
               idx_v, rows_v, acc_dst, acc_rel, sems):
    cid = lax.axis_index("c")
    sid = lax.axis_index("s")
    wid = sid * NC + cid
    rpt = NEP // NS
    rpr = NRP // NS
    cbase = wid * CH

    def start(c, b):
        pltpu.sync_copy(edges3_hbm.at[cbase + c], idx_v.at[b])
        pltpu.async_copy(tab_hbm.at[idx_v.at[b, 0]], rows_v.at[b], sems[b])

    def finish(b):
        pltpu.make_async_copy(tab_hbm.at[idx_v.at[b, 0]], rows_v.at[b],
                              sems[b]).wait()
        pltpu.sync_copy(rows_v.at[b], acc_dst.at[idx_v.at[b, 1]], add=True)
        pltpu.sync_copy(rows_v.at[b], acc_rel.at[idx_v.at[b, 2]], add=True)

    for j in range(NB - 1):
        start(j, j)

    pltpu.sync_copy(zeros_hbm.at[pl.ds(sid * rpt, rpt)],
                    acc_dst.at[pl.ds(sid * rpt, rpt)])
    pltpu.sync_copy(zeros_hbm.at[pl.ds(sid * rpr, rpr)],
                    acc_rel.at[pl.ds(sid * rpr, rpr)])
    plsc.subcore_barrier()

    @pl.loop(0, CH, step=NB)
    def _(c):
        for b in range(NB):
            nc = c + b + NB - 1
            nb = (b + NB - 1) % NB

            @pl.when(nc < CH)
            def _():
                start(nc, nb)

            finish(b)

    plsc.subcore_barrier()
    pltpu.sync_copy(acc_dst.at[pl.ds(sid * rpt, rpt)],
                    out_dst.at[cid, pl.ds(sid * rpt, rpt)])
    pltpu.sync_copy(acc_rel.at[pl.ds(sid * rpr, rpr)],
                    out_rel.at[cid, pl.ds(sid * rpr, rpr)])


def _counts_body(edges3_hbm, zeros_hbm, ones_hbm, out_deg, out_rc,
                 idx_v, ones_v, acc_deg, acc_rc, sems):
    cid = lax.axis_index("c")
    sid = lax.axis_index("s")
    wid = sid * NC + cid
    rpt = NEP // NS
    rpr = NRP // NS
    pltpu.sync_copy(zeros_hbm.at[pl.ds(sid * rpt, rpt)],
                    acc_deg.at[pl.ds(sid * rpt, rpt)])
    pltpu.sync_copy(zeros_hbm.at[pl.ds(sid * rpr, rpr)],
                    acc_rc.at[pl.ds(sid * rpr, rpr)])
    pltpu.sync_copy(ones_hbm, ones_v)
    plsc.subcore_barrier()
    cbase = wid * CH

    def start(c, b):
        pltpu.async_copy(edges3_hbm.at[cbase + c], idx_v.at[b], sems[b])

    def finish(b):
        pltpu.make_async_copy(edges3_hbm.at[cbase], idx_v.at[b], sems[b]).wait()
        pltpu.sync_copy(ones_v, acc_deg.at[idx_v.at[b, 1]], add=True)
        pltpu.sync_copy(ones_v, acc_rc.at[idx_v.at[b, 2]], add=True)

    start(0, 0)

    @pl.loop(0, CH, step=2)
    def _(c):
        start(c + 1, 1)
        finish(0)

        @pl.when(c + 2 < CH)
        def _():
            start(c + 2, 0)

        finish(1)

    plsc.subcore_barrier()
    pltpu.sync_copy(acc_deg.at[pl.ds(sid * rpt, rpt)],
                    out_deg.at[cid, pl.ds(sid * rpt, rpt)])
    pltpu.sync_copy(acc_rc.at[pl.ds(sid * rpr, rpr)],
                    out_rc.at[cid, pl.ds(sid * rpr, rpr)])


@functools.partial(
    pl.kernel,
    out_type=[jax.ShapeDtypeStruct((NC, NEP, H), jnp.float32),
              jax.ShapeDtypeStruct((NC, NRP, H), jnp.float32)],
    mesh=_mesh,
    scratch_types=[
        pltpu.VMEM((NB, 3, K), jnp.int32),
        pltpu.VMEM((NB, K, H), jnp.float32),
        pltpu.VMEM_SHARED((NEP, H), jnp.float32),
        pltpu.VMEM_SHARED((NRP, H), jnp.float32),
    ] + [pltpu.SemaphoreType.DMA] * NB,
)
def _edge_pass(tab_hbm, edges3_hbm, zeros_hbm, out_dst, out_rel,
               idx_v, rows_v, acc_dst, acc_rel, *sems):
    _edge_body(tab_hbm, edges3_hbm, zeros_hbm, out_dst, out_rel,
               idx_v, rows_v, acc_dst, acc_rel, sems)


@functools.partial(
    pl.kernel,
    out_type=[jax.ShapeDtypeStruct((NC, NEP, H), jnp.float32),
              jax.ShapeDtypeStruct((NC, NRP, H), jnp.float32),
              jax.ShapeDtypeStruct((NC, NEP, H), jnp.float32),
              jax.ShapeDtypeStruct((NC, NRP, H), jnp.float32)],
    mesh=_mesh,
    scratch_types=[
        pltpu.VMEM((NB, 3, K), jnp.int32),
        pltpu.VMEM((NB, K, H), jnp.float32),
        pltpu.VMEM_SHARED((NEP, H), jnp.float32),
        pltpu.VMEM_SHARED((NRP, H), jnp.float32),
    ] + [pltpu.SemaphoreType.DMA] * NB,
)
def _edge_counts_pass(tab_hbm, edges3_hbm, zeros_hbm, ones_hbm,
                      out_dst, out_rel, out_deg, out_rc,
                      idx_v, rows_v, acc_dst, acc_rel, *sems):
    _edge_body(tab_hbm, edges3_hbm, zeros_hbm, out_dst, out_rel,
               idx_v, rows_v, acc_dst, acc_rel, sems)
    _counts_body(edges3_hbm, zeros_hbm, ones_hbm, out_deg, out_rc,
                 idx_v, rows_v.at[0], acc_dst, acc_rel, sems)


@functools.partial(
    pl.kernel,
    out_type=[jax.ShapeDtypeStruct((NC, NEP, H), jnp.float32)],
    mesh=_mesh,
    scratch_types=[
        pltpu.VMEM((NB, 3, K), jnp.int32),
        pltpu.VMEM((NB, K, H), jnp.float32),
        pltpu.VMEM_SHARED((NEP, H), jnp.float32),
    ] + [pltpu.SemaphoreType.DMA] * NB,
)
def _rel_pass(r_hbm, edges3_hbm, zeros_hbm, out_dst,
              idx_v, rows_v, acc_dst, *sems):
    cid = lax.axis_index("c")
    sid = lax.axis_index("s")
    wid = sid * NC + cid
    rpt = NEP // NS
    pltpu.sync_copy(zeros_hbm.at[pl.ds(sid * rpt, rpt)],
                    acc_dst.at[pl.ds(sid * rpt, rpt)])
    plsc.subcore_barrier()
    cbase = wid * CH

    def start(c, b):
        pltpu.sync_copy(edges3_hbm.at[cbase + c], idx_v.at[b])
        pltpu.async_copy(r_hbm.at[idx_v.at[b, 2]], rows_v.at[b], sems[b])

    def finish(b):
        pltpu.make_async_copy(r_hbm.at[idx_v.at[b, 2]], rows_v.at[b],
                              sems[b]).wait()
        pltpu.sync_copy(rows_v.at[b], acc_dst.at[idx_v.at[b, 1]], add=True)

    for j in range(NB - 1):
        start(j, j)

    @pl.loop(0, CH, step=NB)
    def _(c):
        for b in range(NB):
            nc = c + b + NB - 1
            nb = (b + NB - 1) % NB

            @pl.when(nc < CH)
            def _():
                start(nc, nb)

            finish(b)

    plsc.subcore_barrier()
    pltpu.sync_copy(acc_dst.at[pl.ds(sid * rpt, rpt)],
                    out_dst.at[cid, pl.ds(sid * rpt, rpt)])


def _norm_body(x_ref, o_ref):
    x = x_ref[...]
    n = jnp.sqrt(jnp.sum(x * x, axis=1, keepdims=True)) + 1e-12
    o_ref[...] = x / n


def _l2norm_rows(x):
    rows = x.shape[0]
    blk = 1024 if rows % 1024 == 0 else rows
    spec = pl.BlockSpec((blk, H), lambda i: (i, 0))
    return pl.pallas_call(
        _norm_body,
        grid=(rows // blk,),
        in_specs=[spec],
        out_specs=spec,
        out_shape=jax.ShapeDtypeStruct((rows, H), jnp.float32),
    )(x)


def _gru(r_agg, r, er, wih, whh, bih, bhh):
    x = jnp.concatenate([r_agg, er], axis=1)
    gi = jnp.dot(x, wih, preferred_element_type=jnp.float32) + bih
    gh = jnp.dot(r, whh, preferred_element_type=jnp.float32) + bhh
    rg = jax.nn.sigmoid(gi[:, 0:H] + gh[:, 0:H])
    zg = jax.nn.sigmoid(gi[:, H:2 * H] + gh[:, H:2 * H])
    ng = jnp.tanh(gi[:, 2 * H:3 * H] + rg * gh[:, 2 * H:3 * H])
    r_new = (1.0 - zg) * ng + zg * r
    n = jnp.sqrt(jnp.sum(r_new * r_new, axis=1, keepdims=True)) + 1e-12
    return r_new / n


def _rel_first_body(sr0_ref, sr1_ref, rc0_ref, rc1_ref, r_ref, er_ref,
                    wih_ref, whh_ref, bih_ref, bhh_ref, out_r, out_invrc):
    invrc = 1.0 / jnp.maximum(rc0_ref[...][:, 0:1] + rc1_ref[...][:, 0:1], 1.0)
    r_agg = (sr0_ref[...] + sr1_ref[...]) * invrc
    rn = _gru(r_agg, r_ref[...], er_ref[...], wih_ref[...],
              whh_ref[...], bih_ref[...], bhh_ref[...])
    out_r[...] = rn
    out_invrc[...] = jnp.broadcast_to(invrc, (NRP, H))


def _rel_rest_body(sr0_ref, sr1_ref, invrc_ref, r_ref, er_ref, wih_ref,
                   whh_ref, bih_ref, bhh_ref, out_r):
    r_agg = (sr0_ref[...] + sr1_ref[...]) * invrc_ref[...]
    out_r[...] = _gru(r_agg, r_ref[...], er_ref[...], wih_ref[...],
                      whh_ref[...], bih_ref[...], bhh_ref[...])


def _rel_first(sr0, sr1, rc0, rc1, r, er, wih_t, whh_t, bih, bhh):
    return pl.pallas_call(
        _rel_first_body,
        out_shape=[jax.ShapeDtypeStruct((NRP, H), jnp.float32),
                   jax.ShapeDtypeStruct((NRP, H), jnp.float32)],
    )(sr0, sr1, rc0, rc1, r, er, wih_t, whh_t, bih, bhh)


def _rel_rest(sr0, sr1, invrc, r, er, wih_t, whh_t, bih, bhh):
    return pl.pallas_call(
        _rel_rest_body,
        out_shape=jax.ShapeDtypeStruct((NRP, H), jnp.float32),
    )(sr0, sr1, invrc, r, er, wih_t, whh_t, bih, bhh)


def _entity(x, h, wn, wl, tw, tb):
    agg = jnp.dot(x, wn, preferred_element_type=jnp.float32)
    loop = jnp.dot(h, wl, preferred_element_type=jnp.float32)
    t = agg + loop
    cur = jnp.where(t >= 0, t, t * SLOPE)
    n = jnp.sqrt(jnp.sum(cur * cur, axis=1, keepdims=True)) + 1e-12
    cur = cur / n
    tg = jax.nn.sigmoid(jnp.dot(cur, tw, preferred_element_type=jnp.float32) + tb)
    return tg * cur + (1.0 - tg) * h


def _h_first_body(sh0_ref, sh1_ref, sr0_ref, sr1_ref, d0_ref, d1_ref, h_ref,
                  wn_ref, wl_ref, tw_ref, tb_ref, out_h, out_invdeg):
    invdeg = 1.0 / jnp.maximum(d0_ref[...][:, 0:1] + d1_ref[...][:, 0:1], 1.0)
    s = sh0_ref[...] + sh1_ref[...] + sr0_ref[...] + sr1_ref[...]
    hn = _entity(s * invdeg, h_ref[...], wn_ref[...], wl_ref[...],
                 tw_ref[...], tb_ref[...])
    out_h[...] = hn
    out_invdeg[...] = jnp.broadcast_to(invdeg, (out_invdeg.shape[0], H))


def _h_rest_body(sh0_ref, sh1_ref, sr0_ref, sr1_ref, invdeg_ref, h_ref,
                 wn_ref, wl_ref, tw_ref, tb_ref, out_h):
    s = sh0_ref[...] + sh1_ref[...] + sr0_ref[...] + sr1_ref[...]
    out_h[...] = _entity(s * invdeg_ref[...], h_ref[...], wn_ref[...],
                         wl_ref[...], tw_ref[...], tb_ref[...])


def _h_first(sh0, sh1, sr0, sr1, d0, d1, h, wn, wl, tw, tb):
    blk = 1024
    row_spec = pl.BlockSpec((blk, H), lambda i: (i, 0))
    w_spec = pl.BlockSpec((H, H), lambda i: (0, 0))
    b_spec = pl.BlockSpec((1, H), lambda i: (0, 0))
    return pl.pallas_call(
        _h_first_body,
        grid=(NEP // blk,),
        in_specs=[row_spec, row_spec, row_spec, row_spec, row_spec, row_spec,
                  row_spec, w_spec, w_spec, w_spec, b_spec],
        out_specs=[row_spec, row_spec],
        out_shape=[jax.ShapeDtypeStruct((NEP, H), jnp.float32),
                   jax.ShapeDtypeStruct((NEP, H), jnp.float32)],
    )(sh0, sh1, sr0, sr1, d0, d1, h, wn, wl, tw, tb)


def _h_rest(sh0, sh1, sr0, sr1, invdeg, h, wn, wl, tw, tb):
    blk = 1024
    row_spec = pl.BlockSpec((blk, H), lambda i: (i, 0))
    w_spec = pl.BlockSpec((H, H), lambda i: (0, 0))
    b_spec = pl.BlockSpec((1, H), lambda i: (0, 0))
    return pl.pallas_call(
        _h_rest_body,
        grid=(NEP // blk,),
        in_specs=[row_spec, row_spec, row_spec, row_spec, row_spec, row_spec,
                  w_spec, w_spec, w_spec, b_spec],
        out_specs=row_spec,
        out_shape=jax.ShapeDtypeStruct((NEP, H), jnp.float32),
    )(sh0, sh1, sr0, sr1, invdeg, h, wn, wl, tw, tb)


def kernel(edge_index, etype, dynamic_emb, emb_rel, w_neigh, w_loop,
           time_gate_weight, time_gate_bias, gru_w_ih, gru_w_hh, gru_b_ih, gru_b_hh):
    pad = EP - E
    src_p = jnp.concatenate([jnp.asarray(edge_index[0], jnp.int32),
                             jnp.zeros((pad,), jnp.int32)])
    dst_p = jnp.concatenate([jnp.asarray(edge_index[1], jnp.int32),
                             jnp.full((pad,), NE, jnp.int32)])
    et_p = jnp.concatenate([jnp.asarray(etype, jnp.int32),
                            jnp.full((pad,), NR, jnp.int32)])
    edges3 = jnp.stack([src_p.reshape(NW, CH, K), dst_p.reshape(NW, CH, K),
                        et_p.reshape(NW, CH, K)], axis=2).reshape(NW * CH, 3, K)

    emb_pad = jnp.concatenate(
        [dynamic_emb, jnp.zeros((NEP - NE, H), jnp.float32)], axis=0)
    er_p = jnp.concatenate(
        [emb_rel, jnp.zeros((NRP - NR, H), jnp.float32)], axis=0)

    zeros_h = jnp.zeros((NEP, H), jnp.float32)
    ones_h = jnp.ones((K, H), jnp.float32)

    wih_t = jnp.transpose(gru_w_ih)
    whh_t = jnp.transpose(gru_w_hh)
    bih = jnp.reshape(gru_b_ih, (1, 3 * H))
    bhh = jnp.reshape(gru_b_hh, (1, 3 * H))
    tb = jnp.reshape(time_gate_bias, (1, H))

    h = _l2norm_rows(emb_pad)

    sdh, srel, deg_p, rc_p = _edge_counts_pass(h, edges3, zeros_h, ones_h)
    r, invrc = _rel_first(srel[0], srel[1], rc_p[0], rc_p[1], er_p, er_p,
                          wih_t, whh_t, bih, bhh)
    (sdr,) = _rel_pass(r, edges3, zeros_h)
    h, invdeg = _h_first(sdh[0], sdh[1], sdr[0], sdr[1],
                         deg_p[0], deg_p[1], h,
                         w_neigh, w_loop, time_gate_weight, tb)

    for _ in range(2):
        sdh, srel = _edge_pass(h, edges3, zeros_h)
        r = _rel_rest(srel[0], srel[1], invrc, r, er_p,
                      wih_t, whh_t, bih, bhh)
        (sdr,) = _rel_pass(r, edges3, zeros_h)
        h = _h_rest(sdh[0], sdh[1], sdr[0], sdr[1], invdeg, h,
                    w_neigh, w_loop, time_gate_weight, tb)
    return h[:NE]


# speedup vs baseline: 1.1270x; 1.0673x over previous
import functools
import jax
import jax.numpy as jnp
from jax import lax
from jax.experimental import pallas as pl
from jax.experimental.pallas import tpu as pltpu
from jax.experimental.pallas import tpu_sc as plsc

H = 128
NE = 10000
NEP = 10240
NR = 460
NRP = 512
E = 320000
NC = 2
NS = 16
NW = NC * NS
K = 80
NB = 4
EPW = 10240
EP = EPW * NW
CH = EPW // K
SLOPE = (1.0 / 8.0 + 1.0 / 3.0) / 2.0

_mesh = plsc.VectorSubcoreMesh(core_axis_name="c", subcore_axis_name="s",
                               num_cores=NC, num_subcores=NS)


def _edge_body(tab_hbm, edges3_hbm, zeros_hbm, out_dst, out_rel,
               idx_v, rows_v, acc_dst, acc_rel, sems):
    cid = lax.axis_index("c")
    sid = lax.axis_index("s")
    wid = sid * NC + cid
    rpt = NEP // NS
    rpr = NRP // NS
    cbase = wid * CH

    def start(c, b):
        pltpu.sync_copy(edges3_hbm.at[cbase + c], idx_v.at[b])
        pltpu.async_copy(tab_hbm.at[idx_v.at[b, 0]], rows_v.at[b], sems[b])

    def finish(b):
        pltpu.make_async_copy(tab_hbm.at[idx_v.at[b, 0]], rows_v.at[b],
                              sems[b]).wait()
        pltpu.sync_copy(rows_v.at[b], acc_dst.at[idx_v.at[b, 1]], add=True)
        pltpu.sync_copy(rows_v.at[b], acc_rel.at[idx_v.at[b, 2]], add=True)

    pltpu.sync_copy(zeros_hbm.at[pl.ds(sid * rpt, rpt)],
                    acc_dst.at[pl.ds(sid * rpt, rpt)])
    pltpu.sync_copy(zeros_hbm.at[pl.ds(sid * rpr, rpr)],
                    acc_rel.at[pl.ds(sid * rpr, rpr)])
    plsc.subcore_barrier()

    for j in range(NB - 1):
        start(j, j)

    @pl.loop(0, CH, step=NB)
    def _(c):
        for b in range(NB):
            nc = c + b + NB - 1
            nb = (b + NB - 1) % NB

            @pl.when(nc < CH)
            def _():
                start(nc, nb)

            finish(b)

    plsc.subcore_barrier()
    pltpu.sync_copy(acc_dst.at[pl.ds(sid * rpt, rpt)],
                    out_dst.at[cid, pl.ds(sid * rpt, rpt)])
    pltpu.sync_copy(acc_rel.at[pl.ds(sid * rpr, rpr)],
                    out_rel.at[cid, pl.ds(sid * rpr, rpr)])


def _counts_body(edges3_hbm, zeros_hbm, ones_hbm, out_deg, out_rc,
                 idx_v, ones_v, acc_deg, acc_rc, sems):
    cid = lax.axis_index("c")
    sid = lax.axis_index("s")
    wid = sid * NC + cid
    rpt = NEP // NS
    rpr = NRP // NS
    pltpu.sync_copy(zeros_hbm.at[pl.ds(sid * rpt, rpt)],
                    acc_deg.at[pl.ds(sid * rpt, rpt)])
    pltpu.sync_copy(zeros_hbm.at[pl.ds(sid * rpr, rpr)],
                    acc_rc.at[pl.ds(sid * rpr, rpr)])
    pltpu.sync_copy(ones_hbm, ones_v)
    plsc.subcore_barrier()
    cbase = wid * CH

    def start(c, b):
        pltpu.async_copy(edges3_hbm.at[cbase + c], idx_v.at[b], sems[b])

    def finish(b):
        pltpu.make_async_copy(edges3_hbm.at[cbase], idx_v.at[b], sems[b]).wait()
        pltpu.sync_copy(ones_v, acc_deg.at[idx_v.at[b, 1]], add=True)
        pltpu.sync_copy(ones_v, acc_rc.at[idx_v.at[b, 2]], add=True)

    start(0, 0)

    @pl.loop(0, CH, step=2)
    def _(c):
        start(c + 1, 1)
        finish(0)

        @pl.when(c + 2 < CH)
        def _():
            start(c + 2, 0)

        finish(1)

    plsc.subcore_barrier()
    pltpu.sync_copy(acc_deg.at[pl.ds(sid * rpt, rpt)],
                    out_deg.at[cid, pl.ds(sid * rpt, rpt)])
    pltpu.sync_copy(acc_rc.at[pl.ds(sid * rpr, rpr)],
                    out_rc.at[cid, pl.ds(sid * rpr, rpr)])


@functools.partial(
    pl.kernel,
    out_type=[jax.ShapeDtypeStruct((NC, NEP, H), jnp.float32),
              jax.ShapeDtypeStruct((NC, NRP, H), jnp.float32)],
    mesh=_mesh,
    scratch_types=[
        pltpu.VMEM((NB, 3, K), jnp.int32),
        pltpu.VMEM((NB, K, H), jnp.float32),
        pltpu.VMEM_SHARED((NEP, H), jnp.float32),
        pltpu.VMEM_SHARED((NRP, H), jnp.float32),
    ] + [pltpu.SemaphoreType.DMA] * NB,
)
def _edge_pass(tab_hbm, edges3_hbm, zeros_hbm, out_dst, out_rel,
               idx_v, rows_v, acc_dst, acc_rel, *sems):
    _edge_body(tab_hbm, edges3_hbm, zeros_hbm, out_dst, out_rel,
               idx_v, rows_v, acc_dst, acc_rel, sems)


@functools.partial(
    pl.kernel,
    out_type=[jax.ShapeDtypeStruct((NC, NEP, H), jnp.float32),
              jax.ShapeDtypeStruct((NC, NRP, H), jnp.float32)],
    mesh=_mesh,
    scratch_types=[
        pltpu.VMEM((2, 3, K), jnp.int32),
        pltpu.VMEM((K, H), jnp.float32),
        pltpu.VMEM_SHARED((NEP, H), jnp.float32),
        pltpu.VMEM_SHARED((NRP, H), jnp.float32),
        pltpu.SemaphoreType.DMA,
        pltpu.SemaphoreType.DMA,
    ],
)
def _counts(edges3_hbm, zeros_hbm, ones_hbm, out_deg, out_rc,
            idx_v, ones_v, acc_deg, acc_rc, sem0, sem1):
    _counts_body(edges3_hbm, zeros_hbm, ones_hbm, out_deg, out_rc,
                 idx_v, ones_v, acc_deg, acc_rc, (sem0, sem1))


@functools.partial(
    pl.kernel,
    out_type=[jax.ShapeDtypeStruct((NC, NEP, H), jnp.float32)],
    mesh=_mesh,
    scratch_types=[
        pltpu.VMEM((NB, 3, K), jnp.int32),
        pltpu.VMEM((NB, K, H), jnp.float32),
        pltpu.VMEM_SHARED((NEP, H), jnp.float32),
    ] + [pltpu.SemaphoreType.DMA] * NB,
)
def _rel_pass(r_hbm, edges3_hbm, zeros_hbm, out_dst,
              idx_v, rows_v, acc_dst, *sems):
    cid = lax.axis_index("c")
    sid = lax.axis_index("s")
    wid = sid * NC + cid
    rpt = NEP // NS
    pltpu.sync_copy(zeros_hbm.at[pl.ds(sid * rpt, rpt)],
                    acc_dst.at[pl.ds(sid * rpt, rpt)])
    plsc.subcore_barrier()
    cbase = wid * CH

    def start(c, b):
        pltpu.sync_copy(edges3_hbm.at[cbase + c], idx_v.at[b])
        pltpu.async_copy(r_hbm.at[idx_v.at[b, 2]], rows_v.at[b], sems[b])

    def finish(b):
        pltpu.make_async_copy(r_hbm.at[idx_v.at[b, 2]], rows_v.at[b],
                              sems[b]).wait()
        pltpu.sync_copy(rows_v.at[b], acc_dst.at[idx_v.at[b, 1]], add=True)

    for j in range(NB - 1):
        start(j, j)

    @pl.loop(0, CH, step=NB)
    def _(c):
        for b in range(NB):
            nc = c + b + NB - 1
            nb = (b + NB - 1) % NB

            @pl.when(nc < CH)
            def _():
                start(nc, nb)

            finish(b)

    plsc.subcore_barrier()
    pltpu.sync_copy(acc_dst.at[pl.ds(sid * rpt, rpt)],
                    out_dst.at[cid, pl.ds(sid * rpt, rpt)])


def _norm_body(x_ref, o_ref):
    x = x_ref[...]
    n = jnp.sqrt(jnp.sum(x * x, axis=1, keepdims=True)) + 1e-12
    o_ref[...] = x / n


def _l2norm_rows(x):
    rows = x.shape[0]
    blk = 1024 if rows % 1024 == 0 else rows
    spec = pl.BlockSpec((blk, H), lambda i: (i, 0))
    return pl.pallas_call(
        _norm_body,
        grid=(rows // blk,),
        in_specs=[spec],
        out_specs=spec,
        out_shape=jax.ShapeDtypeStruct((rows, H), jnp.float32),
    )(x)


def _gru(r_agg, r, er, wih, whh, bih, bhh):
    x = jnp.concatenate([r_agg, er], axis=1)
    gi = jnp.dot(x, wih, preferred_element_type=jnp.float32) + bih
    gh = jnp.dot(r, whh, preferred_element_type=jnp.float32) + bhh
    rg = jax.nn.sigmoid(gi[:, 0:H] + gh[:, 0:H])
    zg = jax.nn.sigmoid(gi[:, H:2 * H] + gh[:, H:2 * H])
    ng = jnp.tanh(gi[:, 2 * H:3 * H] + rg * gh[:, 2 * H:3 * H])
    r_new = (1.0 - zg) * ng + zg * r
    n = jnp.sqrt(jnp.sum(r_new * r_new, axis=1, keepdims=True)) + 1e-12
    return r_new / n


def _rel_first_body(sr0_ref, sr1_ref, rc0_ref, rc1_ref, r_ref, er_ref,
                    wih_ref, whh_ref, bih_ref, bhh_ref, out_r, out_invrc):
    invrc = 1.0 / jnp.maximum(rc0_ref[...][:, 0:1] + rc1_ref[...][:, 0:1], 1.0)
    r_agg = (sr0_ref[...] + sr1_ref[...]) * invrc
    rn = _gru(r_agg, r_ref[...], er_ref[...], wih_ref[...],
              whh_ref[...], bih_ref[...], bhh_ref[...])
    out_r[...] = rn
    out_invrc[...] = jnp.broadcast_to(invrc, (NRP, H))


def _rel_rest_body(sr0_ref, sr1_ref, invrc_ref, r_ref, er_ref, wih_ref,
                   whh_ref, bih_ref, bhh_ref, out_r):
    r_agg = (sr0_ref[...] + sr1_ref[...]) * invrc_ref[...]
    out_r[...] = _gru(r_agg, r_ref[...], er_ref[...], wih_ref[...],
                      whh_ref[...], bih_ref[...], bhh_ref[...])


def _rel_first(sr0, sr1, rc0, rc1, r, er, wih_t, whh_t, bih, bhh):
    return pl.pallas_call(
        _rel_first_body,
        out_shape=[jax.ShapeDtypeStruct((NRP, H), jnp.float32),
                   jax.ShapeDtypeStruct((NRP, H), jnp.float32)],
    )(sr0, sr1, rc0, rc1, r, er, wih_t, whh_t, bih, bhh)


def _rel_rest(sr0, sr1, invrc, r, er, wih_t, whh_t, bih, bhh):
    return pl.pallas_call(
        _rel_rest_body,
        out_shape=jax.ShapeDtypeStruct((NRP, H), jnp.float32),
    )(sr0, sr1, invrc, r, er, wih_t, whh_t, bih, bhh)


def _entity(x, h, wn, wl, tw, tb):
    agg = jnp.dot(x, wn, preferred_element_type=jnp.float32)
    loop = jnp.dot(h, wl, preferred_element_type=jnp.float32)
    t = agg + loop
    cur = jnp.where(t >= 0, t, t * SLOPE)
    n = jnp.sqrt(jnp.sum(cur * cur, axis=1, keepdims=True)) + 1e-12
    cur = cur / n
    tg = jax.nn.sigmoid(jnp.dot(cur, tw, preferred_element_type=jnp.float32) + tb)
    return tg * cur + (1.0 - tg) * h


def _h_first_body(sh0_ref, sh1_ref, sr0_ref, sr1_ref, d0_ref, d1_ref, h_ref,
                  wn_ref, wl_ref, tw_ref, tb_ref, out_h, out_invdeg):
    invdeg = 1.0 / jnp.maximum(d0_ref[...][:, 0:1] + d1_ref[...][:, 0:1], 1.0)
    s = sh0_ref[...] + sh1_ref[...] + sr0_ref[...] + sr1_ref[...]
    hn = _entity(s * invdeg, h_ref[...], wn_ref[...], wl_ref[...],
                 tw_ref[...], tb_ref[...])
    out_h[...] = hn
    out_invdeg[...] = jnp.broadcast_to(invdeg, (out_invdeg.shape[0], H))


def _h_rest_body(sh0_ref, sh1_ref, sr0_ref, sr1_ref, invdeg_ref, h_ref,
                 wn_ref, wl_ref, tw_ref, tb_ref, out_h):
    s = sh0_ref[...] + sh1_ref[...] + sr0_ref[...] + sr1_ref[...]
    out_h[...] = _entity(s * invdeg_ref[...], h_ref[...], wn_ref[...],
                         wl_ref[...], tw_ref[...], tb_ref[...])


def _h_first(sh0, sh1, sr0, sr1, d0, d1, h, wn, wl, tw, tb):
    blk = 1024
    row_spec = pl.BlockSpec((blk, H), lambda i: (i, 0))
    w_spec = pl.BlockSpec((H, H), lambda i: (0, 0))
    b_spec = pl.BlockSpec((1, H), lambda i: (0, 0))
    return pl.pallas_call(
        _h_first_body,
        grid=(NEP // blk,),
        in_specs=[row_spec, row_spec, row_spec, row_spec, row_spec, row_spec,
                  row_spec, w_spec, w_spec, w_spec, b_spec],
        out_specs=[row_spec, row_spec],
        out_shape=[jax.ShapeDtypeStruct((NEP, H), jnp.float32),
                   jax.ShapeDtypeStruct((NEP, H), jnp.float32)],
    )(sh0, sh1, sr0, sr1, d0, d1, h, wn, wl, tw, tb)


def _h_rest(sh0, sh1, sr0, sr1, invdeg, h, wn, wl, tw, tb):
    blk = 1024
    row_spec = pl.BlockSpec((blk, H), lambda i: (i, 0))
    w_spec = pl.BlockSpec((H, H), lambda i: (0, 0))
    b_spec = pl.BlockSpec((1, H), lambda i: (0, 0))
    return pl.pallas_call(
        _h_rest_body,
        grid=(NEP // blk,),
        in_specs=[row_spec, row_spec, row_spec, row_spec, row_spec, row_spec,
                  w_spec, w_spec, w_spec, b_spec],
        out_specs=row_spec,
        out_shape=jax.ShapeDtypeStruct((NEP, H), jnp.float32),
    )(sh0, sh1, sr0, sr1, invdeg, h, wn, wl, tw, tb)


def kernel(edge_index, etype, dynamic_emb, emb_rel, w_neigh, w_loop,
           time_gate_weight, time_gate_bias, gru_w_ih, gru_w_hh, gru_b_ih, gru_b_hh):
    pad = EP - E
    src_p = jnp.concatenate([jnp.asarray(edge_index[0], jnp.int32),
                             jnp.zeros((pad,), jnp.int32)])
    dst_p = jnp.concatenate([jnp.asarray(edge_index[1], jnp.int32),
                             jnp.full((pad,), NE, jnp.int32)])
    et_p = jnp.concatenate([jnp.asarray(etype, jnp.int32),
                            jnp.full((pad,), NR, jnp.int32)])
    edges3 = jnp.stack([src_p.reshape(NW, CH, K), dst_p.reshape(NW, CH, K),
                        et_p.reshape(NW, CH, K)], axis=2).reshape(NW * CH, 3, K)

    emb_pad = jnp.concatenate(
        [dynamic_emb, jnp.zeros((NEP - NE, H), jnp.float32)], axis=0)
    er_p = jnp.concatenate(
        [emb_rel, jnp.zeros((NRP - NR, H), jnp.float32)], axis=0)

    zeros_h = jnp.zeros((NEP, H), jnp.float32)
    ones_h = jnp.ones((K, H), jnp.float32)

    wih_t = jnp.transpose(gru_w_ih)
    whh_t = jnp.transpose(gru_w_hh)
    bih = jnp.reshape(gru_b_ih, (1, 3 * H))
    bhh = jnp.reshape(gru_b_hh, (1, 3 * H))
    tb = jnp.reshape(time_gate_bias, (1, H))

    deg_p, rc_p = _counts(edges3, zeros_h, ones_h)
    h = _l2norm_rows(emb_pad)

    sdh, srel = _edge_pass(h, edges3, zeros_h)
    r, invrc = _rel_first(srel[0], srel[1], rc_p[0], rc_p[1], er_p, er_p,
                          wih_t, whh_t, bih, bhh)
    (sdr,) = _rel_pass(r, edges3, zeros_h)
    h, invdeg = _h_first(sdh[0], sdh[1], sdr[0], sdr[1],
                         deg_p[0], deg_p[1], h,
                         w_neigh, w_loop, time_gate_weight, tb)

    for _ in range(2):
        sdh, srel = _edge_pass(h, edges3, zeros_h)
        r = _rel_rest(srel[0], srel[1], invrc, r, er_p,
                      wih_t, whh_t, bih, bhh)
        (sdr,) = _rel_pass(r, edges3, zeros_h)
        h = _h_rest(sdh[0], sdh[1], sdr[0], sdr[1], invdeg, h,
                    w_neigh, w_loop, time_gate_weight, tb)
    return h[:NE]
